# Initial kernel scaffold; baseline (speedup 1.0000x reference)
import functools

import jax
import jax.numpy as jnp
from jax import lax
from jax.experimental import pallas as pl
from jax.experimental.pallas import tpu as pltpu
from jax.experimental.pallas import tpu_sc as plsc

_PRIMES = [100003, 100019, 100043, 100049, 100057, 100069, 100103, 100109]
_ED = 16
_LANES = 16
_NC = 2
_NS = 16
_NW = _NC * _NS

_STREAM = 128
_K = 20


def _head_offsets16():
    offs = [0]
    for p in _PRIMES[:-1]:
        offs.append(offs[-1] + p)
    return jnp.asarray(offs + offs, dtype=jnp.int32)


def _make_kernel(n_rows):
    rows_per_w = n_rows // _NW
    n_chunks = rows_per_w // _K
    mesh = plsc.VectorSubcoreMesh(core_axis_name="c", subcore_axis_name="s")

    @functools.partial(
        pl.kernel,
        mesh=mesh,
        out_type=jax.ShapeDtypeStruct((n_rows, _STREAM, _ED), jnp.float32),
        scratch_types=[
            pltpu.VMEM((_K, _STREAM), jnp.int32),
            pltpu.VMEM((_K, _STREAM, _ED), jnp.float32),
            pltpu.VMEM((_LANES,), jnp.int32),
            pltpu.SemaphoreType.DMA,
        ],
    )
    def k(w_hbm, idx_hbm, off_hbm, out_hbm, idx_v, rows_v, off_v, sem):
        wid = lax.axis_index("s") * _NC + lax.axis_index("c")
        row0 = wid * rows_per_w
        pltpu.sync_copy(off_hbm, off_v)
        off = off_v[...]

        def chunk_body(g, carry):
            base = row0 + g * _K
            pltpu.sync_copy(idx_hbm.at[pl.ds(base, _K)], idx_v)

            def add_offs(j, c):
                for i in range(_STREAM // _LANES):
                    sl = (j, pl.ds(i * _LANES, _LANES))
                    idx_v[sl] = idx_v[sl] + off
                return c

            lax.fori_loop(0, _K, add_offs, 0)

            copies = [
                pltpu.async_copy(w_hbm.at[idx_v.at[j]], rows_v.at[j], sem)
                for j in range(_K)
            ]
            for c in copies:
                c.wait()

            pltpu.sync_copy(rows_v, out_hbm.at[pl.ds(base, _K)])
            return carry

        lax.fori_loop(0, n_chunks, chunk_body, 0)

    return k


def kernel(hash_indices, weight):
    b, s, h = hash_indices.shape
    n = b * s * h
    n_rows = n // _STREAM
    idx2d = hash_indices.reshape(n_rows, _STREAM).astype(jnp.int32)
    out = _make_kernel(n_rows)(weight, idx2d, _head_offsets16())
    return out.reshape(b, s, h, _ED)


# speedup vs baseline: 22.7316x; 22.7316x over previous
import functools

import jax
import jax.numpy as jnp
from jax import lax
from jax.experimental import pallas as pl
from jax.experimental.pallas import tpu as pltpu
from jax.experimental.pallas import tpu_sc as plsc

_PRIMES = [100003, 100019, 100043, 100049, 100057, 100069, 100103, 100109]
_ED = 16
_LANES = 16
_NC = 2
_NS = 16
_NW = _NC * _NS

_STREAM = 128
_K = 16


def _head_offsets16():
    offs = [0]
    for p in _PRIMES[:-1]:
        offs.append(offs[-1] + p)
    return jnp.asarray(offs + offs, dtype=jnp.int32)


def _make_kernel(n_rows):
    rows_per_w = n_rows // _NW
    n_chunks = rows_per_w // _K
    mesh = plsc.VectorSubcoreMesh(core_axis_name="c", subcore_axis_name="s")

    @functools.partial(
        pl.kernel,
        mesh=mesh,
        compiler_params=pltpu.CompilerParams(use_tc_tiling_on_sc=False),
        out_type=jax.ShapeDtypeStruct((n_rows, _STREAM, _ED), jnp.float32),
        scratch_types=[
            pltpu.VMEM((_K, _STREAM), jnp.int32),
            pltpu.VMEM((_K, _STREAM, _ED), jnp.float32),
            pltpu.VMEM((_LANES,), jnp.int32),
            pltpu.SemaphoreType.DMA,
        ],
    )
    def k(w_hbm, idx_hbm, off_hbm, out_hbm, idx_v, rows_v, off_v, sem):
        wid = lax.axis_index("s") * _NC + lax.axis_index("c")
        row0 = wid * rows_per_w
        pltpu.sync_copy(off_hbm, off_v)
        off = off_v[...]

        def chunk_body(g, carry):
            base = row0 + g * _K
            pltpu.sync_copy(idx_hbm.at[pl.ds(base, _K)], idx_v)

            def add_offs(j, c):
                for i in range(_STREAM // _LANES):
                    sl = (j, pl.ds(i * _LANES, _LANES))
                    idx_v[sl] = idx_v[sl] + off
                return c

            lax.fori_loop(0, _K, add_offs, 0)

            copies = [
                pltpu.async_copy(w_hbm.at[idx_v.at[j]], rows_v.at[j], sem)
                for j in range(_K)
            ]
            for c in copies:
                c.wait()

            pltpu.sync_copy(rows_v, out_hbm.at[pl.ds(base, _K)])
            return carry

        lax.fori_loop(0, n_chunks, chunk_body, 0)

    return k


def kernel(hash_indices, weight):
    b, s, h = hash_indices.shape
    n = b * s * h
    n_rows = n // _STREAM
    idx2d = hash_indices.reshape(n_rows, _STREAM).astype(jnp.int32)
    out = _make_kernel(n_rows)(weight, idx2d, _head_offsets16())
    return out.reshape(b, s, h, _ED)
